# baseline (speedup 1.0000x reference)
import functools

import jax
import jax.numpy as jnp
from jax import lax
from jax.experimental import pallas as pl
from jax.experimental.pallas import tpu as pltpu
from jax.experimental.pallas import tpu_sc as plsc

N = 10000
D = 128
E = 320000
NC, NS = 2, 16
NW = NC * NS
EPT = E // NW
CHUNK = 128
NCH = 80
NP = 10240
RPT = NP // NS
FPAD = 256

_mesh = plsc.VectorSubcoreMesh(core_axis_name="c", subcore_axis_name="s")


@functools.partial(
    pl.kernel,
    out_type=(jax.ShapeDtypeStruct((NC, NP, D), jnp.float32),
              jax.ShapeDtypeStruct((NW * NCH * CHUNK + FPAD, D), jnp.float32)),
    mesh=_mesh,
    scratch_types=[
        pltpu.VMEM_SHARED((NP, D), jnp.float32),
        pltpu.VMEM((NCH, CHUNK), jnp.int32),
        pltpu.VMEM((CHUNK, D), jnp.float32),
        pltpu.VMEM((CHUNK, D), jnp.float32),
        pltpu.SemaphoreType.DMA,
        pltpu.SemaphoreType.DMA,
        pltpu.SemaphoreType.DMA,
        pltpu.SemaphoreType.DMA,
    ],
)
def _scatter_sum(table_hbm, src_hbm, dst_hbm, out_hbm, feat_hbm,
                 sp, ib, rows0, rows1, g0, g1, w0, w1):
    c = lax.axis_index("c")
    s = lax.axis_index("s")
    wid = c * NS + s
    r0 = s * RPT
    fbase = wid * (NCH * CHUNK)
    rows = (rows0, rows1)
    gsem = (g0, g1)
    wsem = (w0, w1)

    def feat(j):
        return feat_hbm.at[pl.ds(fbase + j * CHUNK, CHUNK)]

    pltpu.sync_copy(src_hbm.at[c, s], ib)

    @pl.when(s < NS - 1)
    def _stage_full():
        pltpu.sync_copy(table_hbm.at[pl.ds(r0, RPT)], sp.at[pl.ds(r0, RPT)])

    @pl.when(s == NS - 1)
    def _stage_tail():
        pltpu.sync_copy(table_hbm.at[pl.ds(r0, N - (NS - 1) * RPT)],
                        sp.at[pl.ds(r0, N - (NS - 1) * RPT)])

    plsc.subcore_barrier()

    pltpu.async_copy(rows0, feat(0), w0)
    pltpu.async_copy(rows1, feat(1), w1)

    def _p1(k, carry):
        j = 2 * k
        pltpu.make_async_copy(rows0, feat(j), w0).wait()
        pltpu.async_copy(sp.at[ib.at[j]], rows0, g0)
        pltpu.make_async_copy(rows1, feat(j + 1), w1).wait()
        pltpu.async_copy(sp.at[ib.at[j + 1]], rows1, g1)
        pltpu.make_async_copy(sp.at[ib.at[j]], rows0, g0).wait()
        pltpu.async_copy(rows0, feat(j), w0)
        pltpu.make_async_copy(sp.at[ib.at[j + 1]], rows1, g1).wait()
        pltpu.async_copy(rows1, feat(j + 1), w1)
        return carry

    lax.fori_loop(0, NCH // 2, _p1, 0)
    pltpu.make_async_copy(rows0, feat(NCH - 2), w0).wait()
    pltpu.make_async_copy(rows1, feat(NCH - 1), w1).wait()

    plsc.subcore_barrier()

    pltpu.sync_copy(dst_hbm.at[c, s], ib)
    zero = jnp.zeros((16,), jnp.float32)

    def _zrow(i, carry):
        for q in range(D // 16):
            rows0[i, pl.ds(q * 16, 16)] = zero
        return carry

    lax.fori_loop(0, CHUNK, _zrow, 0)
    for off in range(0, RPT, CHUNK):
        pltpu.sync_copy(rows0, sp.at[pl.ds(r0 + off, CHUNK)])

    plsc.subcore_barrier()

    pltpu.async_copy(feat(0), rows0, g0)
    pltpu.async_copy(feat(1), rows1, g1)

    def _p2(k, carry):
        j = 2 * k
        pltpu.make_async_copy(feat(j), rows0, g0).wait()
        pltpu.async_copy(rows0, sp.at[ib.at[j]], w0, add=True)
        pltpu.make_async_copy(feat(j + 1), rows1, g1).wait()
        pltpu.async_copy(rows1, sp.at[ib.at[j + 1]], w1, add=True)
        pltpu.make_async_copy(rows0, sp.at[ib.at[j]], w0).wait()
        pltpu.async_copy(feat(j + 2), rows0, g0)
        pltpu.make_async_copy(rows1, sp.at[ib.at[j + 1]], w1).wait()
        pltpu.async_copy(feat(j + 3), rows1, g1)
        return carry

    lax.fori_loop(0, NCH // 2, _p2, 0)
    pltpu.make_async_copy(feat(NCH), rows0, g0).wait()
    pltpu.make_async_copy(feat(NCH + 1), rows1, g1).wait()

    plsc.subcore_barrier()

    pltpu.sync_copy(sp.at[pl.ds(r0, RPT)], out_hbm.at[c, pl.ds(r0, RPT)])


def _add_block(a_ref, b_ref, o_ref):
    o_ref[...] = a_ref[0] + b_ref[0]


_combine = pl.pallas_call(
    _add_block,
    grid=(10,),
    in_specs=[pl.BlockSpec((1, N // 10, D), lambda i: (0, i, 0)),
              pl.BlockSpec((1, N // 10, D), lambda i: (1, i, 0))],
    out_specs=pl.BlockSpec((N // 10, D), lambda i: (i, 0)),
    out_shape=jax.ShapeDtypeStruct((N, D), jnp.float32),
)


def kernel(face_x, face_index):
    src = face_index[0].astype(jnp.int32).reshape(NC, NS, EPT)
    dst = face_index[1].astype(jnp.int32).reshape(NC, NS, EPT)
    pad = NCH * CHUNK - EPT
    src = jnp.pad(src, ((0, 0), (0, 0), (0, pad))).reshape(NC, NS, NCH, CHUNK)
    dst = jnp.pad(dst, ((0, 0), (0, 0), (0, pad)),
                  constant_values=N).reshape(NC, NS, NCH, CHUNK)
    y, _ = _scatter_sum(face_x, src, dst)
    return _combine(y, y)


# speedup vs baseline: 1.3245x; 1.3245x over previous
import functools

import jax
import jax.numpy as jnp
from jax import lax
from jax.experimental import pallas as pl
from jax.experimental.pallas import tpu as pltpu
from jax.experimental.pallas import tpu_sc as plsc

N = 10000
D = 128
E = 320000
NC, NS = 2, 16
NW = NC * NS
EPT = E // NW
CHUNK = 120
BLK = 6
NBLK = 14
NCH = BLK * NBLK
NP = 10112
RPT = NP // NS
SLAB = NCH * CHUNK
FPAD = 256

_mesh = plsc.VectorSubcoreMesh(core_axis_name="c", subcore_axis_name="s")


@functools.partial(
    pl.kernel,
    out_type=(jax.ShapeDtypeStruct((NC, NP, D), jnp.float32),
              jax.ShapeDtypeStruct((NW * SLAB + FPAD, D), jnp.float32)),
    mesh=_mesh,
    scratch_types=[
        pltpu.VMEM_SHARED((NP, D), jnp.float32),
        pltpu.VMEM((2 * BLK, CHUNK), jnp.int32),
        pltpu.VMEM((CHUNK, D), jnp.float32),
        pltpu.VMEM((CHUNK, D), jnp.float32),
        pltpu.VMEM((CHUNK, D), jnp.float32),
        pltpu.SemaphoreType.DMA,
        pltpu.SemaphoreType.DMA,
        pltpu.SemaphoreType.DMA,
        pltpu.SemaphoreType.DMA,
        pltpu.SemaphoreType.DMA,
        pltpu.SemaphoreType.DMA,
    ],
)
def _scatter_sum(table_hbm, src_hbm, dst_hbm, out_hbm, feat_hbm,
                 sp, ib, rows0, rows1, rows2, g0, g1, g2, w0, w1, w2):
    c = lax.axis_index("c")
    s = lax.axis_index("s")
    wid = c * NS + s
    r0 = s * RPT
    fbase = wid * SLAB
    rows = (rows0, rows1, rows2)
    g = (g0, g1, g2)
    w = (w0, w1, w2)

    def feat(j):
        return feat_hbm.at[pl.ds(fbase + j * CHUNK, CHUNK)]

    @pl.when(s < NS - 1)
    def _stage_full():
        pltpu.sync_copy(table_hbm.at[pl.ds(r0, RPT)], sp.at[pl.ds(r0, RPT)])

    @pl.when(s == NS - 1)
    def _stage_tail():
        pltpu.sync_copy(table_hbm.at[pl.ds(r0, N - (NS - 1) * RPT)],
                        sp.at[pl.ds(r0, N - (NS - 1) * RPT)])

    plsc.subcore_barrier()

    for i in range(3):
        pltpu.async_copy(rows[i], feat(i), w[i])

    def _p1blk(blk, carry):
        h = (blk % 2) * BLK
        pltpu.sync_copy(src_hbm.at[c, s, blk], ib.at[pl.ds(h, BLK)])
        j0 = blk * BLK
        for u in range(BLK):
            b = u % 3
            pltpu.make_async_copy(rows[b], feat(0), w[b]).wait()
            pltpu.async_copy(sp.at[ib.at[h + u]], rows[b], g[b])
            bw = (u + 1) % 3
            if u < 2:
                @pl.when(blk > 0)
                def _wlag():
                    pltpu.make_async_copy(sp.at[ib.at[h + u]],
                                          rows[bw], g[bw]).wait()
                    pltpu.async_copy(rows[bw], feat(j0 + u - 2), w[bw])
            else:
                pltpu.make_async_copy(sp.at[ib.at[h + u]],
                                      rows[bw], g[bw]).wait()
                pltpu.async_copy(rows[bw], feat(j0 + u - 2), w[bw])
        return carry

    lax.fori_loop(0, NBLK, _p1blk, 0)
    pltpu.make_async_copy(rows[(NCH - 2) % 3], feat(0), g[(NCH - 2) % 3]).wait()
    pltpu.async_copy(rows[(NCH - 2) % 3], feat(NCH - 2), w[(NCH - 2) % 3])
    pltpu.make_async_copy(rows[(NCH - 1) % 3], feat(0), g[(NCH - 1) % 3]).wait()
    pltpu.async_copy(rows[(NCH - 1) % 3], feat(NCH - 1), w[(NCH - 1) % 3])
    for i in range(3):
        pltpu.make_async_copy(rows[i], feat(0), w[i]).wait()

    plsc.subcore_barrier()

    zero = jnp.zeros((16,), jnp.float32)

    def _zrow(i, carry):
        for q in range(D // 16):
            rows0[i, pl.ds(q * 16, 16)] = zero
        return carry

    lax.fori_loop(0, CHUNK, _zrow, 0)
    for off in (0, CHUNK, 2 * CHUNK, 3 * CHUNK, 4 * CHUNK, RPT - CHUNK):
        pltpu.sync_copy(rows0, sp.at[pl.ds(r0 + off, CHUNK)])

    plsc.subcore_barrier()

    pltpu.async_copy(feat(0), rows0, g0)

    def _p2blk(blk, carry):
        h = (blk % 2) * BLK
        pltpu.sync_copy(dst_hbm.at[c, s, blk], ib.at[pl.ds(h, BLK)])
        j0 = blk * BLK
        for u in range(BLK):
            b = u % 3
            pltpu.async_copy(feat(j0 + u + 1), rows[(u + 1) % 3],
                             g[(u + 1) % 3])
            pltpu.make_async_copy(feat(j0 + u), rows[b], g[b]).wait()
            pltpu.sync_copy(rows[b], sp.at[ib.at[h + u]], add=True)
        return carry

    lax.fori_loop(0, NBLK, _p2blk, 0)
    pltpu.make_async_copy(feat(NCH), rows[NCH % 3], g[NCH % 3]).wait()

    plsc.subcore_barrier()

    pltpu.sync_copy(sp.at[pl.ds(r0, RPT)], out_hbm.at[c, pl.ds(r0, RPT)])


def _add_block(a_ref, b_ref, o_ref):
    o_ref[...] = a_ref[0] + b_ref[0]


_combine = pl.pallas_call(
    _add_block,
    grid=(10,),
    in_specs=[pl.BlockSpec((1, N // 10, D), lambda i: (0, i, 0)),
              pl.BlockSpec((1, N // 10, D), lambda i: (1, i, 0))],
    out_specs=pl.BlockSpec((N // 10, D), lambda i: (i, 0)),
    out_shape=jax.ShapeDtypeStruct((N, D), jnp.float32),
)


def kernel(face_x, face_index):
    src = face_index[0].astype(jnp.int32).reshape(NC, NS, EPT)
    dst = face_index[1].astype(jnp.int32).reshape(NC, NS, EPT)
    pad = SLAB - EPT
    src = jnp.pad(src, ((0, 0), (0, 0), (0, pad))
                  ).reshape(NC, NS, NBLK, BLK, CHUNK)
    dst = jnp.pad(dst, ((0, 0), (0, 0), (0, pad)),
                  constant_values=N).reshape(NC, NS, NBLK, BLK, CHUNK)
    y, _ = _scatter_sum(face_x, src, dst)
    return _combine(y, y)


# speedup vs baseline: 1.4075x; 1.0627x over previous
import functools

import jax
import jax.numpy as jnp
from jax import lax
from jax.experimental import pallas as pl
from jax.experimental.pallas import tpu as pltpu
from jax.experimental.pallas import tpu_sc as plsc

N = 10000
D = 128
E = 320000
NC, NS = 2, 16
NW = NC * NS
EPT = E // NW
CHUNK = 120
BLK = 6
NBLK = 14
NCH = BLK * NBLK
NP = 10112
RPT = NP // NS
SLAB = NCH * CHUNK
FPAD = 256

_mesh = plsc.VectorSubcoreMesh(core_axis_name="c", subcore_axis_name="s")


@functools.partial(
    pl.kernel,
    out_type=(jax.ShapeDtypeStruct((NC, NP, D), jnp.float32),
              jax.ShapeDtypeStruct((NW * SLAB + FPAD, D), jnp.float32)),
    mesh=_mesh,
    scratch_types=[
        pltpu.VMEM_SHARED((NP, D), jnp.float32),
        pltpu.VMEM((2 * BLK, CHUNK), jnp.int32),
        pltpu.VMEM((CHUNK, D), jnp.float32),
        pltpu.VMEM((CHUNK, D), jnp.float32),
        pltpu.VMEM((CHUNK, D), jnp.float32),
        pltpu.SemaphoreType.DMA,
        pltpu.SemaphoreType.DMA,
        pltpu.SemaphoreType.DMA,
        pltpu.SemaphoreType.DMA,
        pltpu.SemaphoreType.DMA,
        pltpu.SemaphoreType.DMA,
    ],
)
def _scatter_sum(table_hbm, src_hbm, dst_hbm, out_hbm, feat_hbm,
                 sp, ib, rows0, rows1, rows2, g0, g1, g2, w0, w1, w2):
    c = lax.axis_index("c")
    s = lax.axis_index("s")
    wid = c * NS + s
    r0 = s * RPT
    fbase = wid * SLAB
    rows = (rows0, rows1, rows2)
    g = (g0, g1, g2)
    w = (w0, w1, w2)

    def feat(j):
        return feat_hbm.at[pl.ds(fbase + j * CHUNK, CHUNK)]

    @pl.when(s < NS - 1)
    def _stage_full():
        pltpu.sync_copy(table_hbm.at[pl.ds(r0, RPT)], sp.at[pl.ds(r0, RPT)])

    @pl.when(s == NS - 1)
    def _stage_tail():
        pltpu.sync_copy(table_hbm.at[pl.ds(r0, N - (NS - 1) * RPT)],
                        sp.at[pl.ds(r0, N - (NS - 1) * RPT)])

    plsc.subcore_barrier()

    for i in range(3):
        pltpu.async_copy(rows[i], feat(i), w[i])

    def _p1blk(blk, carry):
        h = (blk % 2) * BLK
        pltpu.sync_copy(src_hbm.at[c, s, blk], ib.at[pl.ds(h, BLK)])
        j0 = blk * BLK
        for u in range(BLK):
            b = u % 3
            pltpu.make_async_copy(rows[b], feat(0), w[b]).wait()
            pltpu.async_copy(sp.at[ib.at[h + u]], rows[b], g[b])
            bw = (u + 1) % 3
            if u < 2:
                @pl.when(blk > 0)
                def _wlag():
                    pltpu.make_async_copy(sp.at[ib.at[h + u]],
                                          rows[bw], g[bw]).wait()
                    pltpu.async_copy(rows[bw], feat(j0 + u - 2), w[bw])
            else:
                pltpu.make_async_copy(sp.at[ib.at[h + u]],
                                      rows[bw], g[bw]).wait()
                pltpu.async_copy(rows[bw], feat(j0 + u - 2), w[bw])
        return carry

    lax.fori_loop(0, NBLK, _p1blk, 0)
    pltpu.make_async_copy(rows[(NCH - 2) % 3], feat(0), g[(NCH - 2) % 3]).wait()
    pltpu.async_copy(rows[(NCH - 2) % 3], feat(NCH - 2), w[(NCH - 2) % 3])
    pltpu.make_async_copy(rows[(NCH - 1) % 3], feat(0), g[(NCH - 1) % 3]).wait()
    pltpu.async_copy(rows[(NCH - 1) % 3], feat(NCH - 1), w[(NCH - 1) % 3])
    for i in range(3):
        pltpu.make_async_copy(rows[i], feat(0), w[i]).wait()

    plsc.subcore_barrier()

    zero = jnp.zeros((16,), jnp.float32)

    def _zrow(i, carry):
        for q in range(D // 16):
            rows0[i, pl.ds(q * 16, 16)] = zero
        return carry

    lax.fori_loop(0, CHUNK, _zrow, 0)
    for off in (0, CHUNK, 2 * CHUNK, 3 * CHUNK, 4 * CHUNK, RPT - CHUNK):
        pltpu.sync_copy(rows0, sp.at[pl.ds(r0 + off, CHUNK)])

    plsc.subcore_barrier()

    pltpu.async_copy(feat(0), rows0, g0)
    pltpu.async_copy(feat(1), rows1, g1)

    def _p2blk(blk, carry):
        h = (blk % 2) * BLK
        pltpu.sync_copy(dst_hbm.at[c, s, blk], ib.at[pl.ds(h, BLK)])
        j0 = blk * BLK
        for u in range(BLK):
            b = u % 3
            pltpu.async_copy(feat(j0 + u + 2), rows[(u + 2) % 3],
                             g[(u + 2) % 3])
            pltpu.make_async_copy(feat(j0 + u), rows[b], g[b]).wait()
            pltpu.sync_copy(rows[b], sp.at[ib.at[h + u]], add=True)
        return carry

    lax.fori_loop(0, NBLK, _p2blk, 0)
    pltpu.make_async_copy(feat(NCH), rows[NCH % 3], g[NCH % 3]).wait()
    pltpu.make_async_copy(feat(NCH + 1), rows[(NCH + 1) % 3],
                          g[(NCH + 1) % 3]).wait()

    plsc.subcore_barrier()

    pltpu.sync_copy(sp.at[pl.ds(r0, RPT)], out_hbm.at[c, pl.ds(r0, RPT)])


def _add_block(a_ref, b_ref, o_ref):
    o_ref[...] = a_ref[0] + b_ref[0]


_combine = pl.pallas_call(
    _add_block,
    grid=(10,),
    in_specs=[pl.BlockSpec((1, N // 10, D), lambda i: (0, i, 0)),
              pl.BlockSpec((1, N // 10, D), lambda i: (1, i, 0))],
    out_specs=pl.BlockSpec((N // 10, D), lambda i: (i, 0)),
    out_shape=jax.ShapeDtypeStruct((N, D), jnp.float32),
)


def kernel(face_x, face_index):
    src = face_index[0].astype(jnp.int32).reshape(NC, NS, EPT)
    dst = face_index[1].astype(jnp.int32).reshape(NC, NS, EPT)
    pad = SLAB - EPT
    src = jnp.pad(src, ((0, 0), (0, 0), (0, pad))
                  ).reshape(NC, NS, NBLK, BLK, CHUNK)
    dst = jnp.pad(dst, ((0, 0), (0, 0), (0, pad)),
                  constant_values=N).reshape(NC, NS, NBLK, BLK, CHUNK)
    y, _ = _scatter_sum(face_x, src, dst)
    return _combine(y, y)
